# Initial kernel scaffold; baseline (speedup 1.0000x reference)
import jax
import jax.numpy as jnp
from jax.experimental import pallas as pl


def kernel(h, edge_index, coord, W_e1, b_e1, W_e2, b_e2, W_n1, b_n1, W_n2, b_n2):
    raise NotImplementedError("write your pallas kernel here")


# speedup vs baseline: 4.6975x; 4.6975x over previous
import functools

import jax
import jax.numpy as jnp
from jax import lax
from jax.experimental import pallas as pl
from jax.experimental.pallas import tpu as pltpu
from jax.experimental.pallas import tpu_sc as plsc

NUM_CORES = 2
NUM_SUBCORES = 16
NW = NUM_CORES * NUM_SUBCORES
LANES = 16
CHUNK = 128


def _sc_mesh():
    return plsc.VectorSubcoreMesh(
        core_axis_name="c", subcore_axis_name="s",
        num_cores=NUM_CORES, num_subcores=NUM_SUBCORES)


def _pre_body(h_ref, w1s_ref, w1t_ref, b1_ref, a_ref, b_ref):
    h = h_ref[...]
    a_ref[...] = jnp.dot(h, w1s_ref[...], preferred_element_type=jnp.float32) + b1_ref[...]
    b_ref[...] = jnp.dot(h, w1t_ref[...], preferred_element_type=jnp.float32)


def _precompute(h, w1s, w1t, b1, block=2000):
    n, d = h.shape
    hh = w1s.shape[1]
    grid = n // block
    return pl.pallas_call(
        _pre_body,
        grid=(grid,),
        in_specs=[
            pl.BlockSpec((block, d), lambda i: (i, 0)),
            pl.BlockSpec((d, hh), lambda i: (0, 0)),
            pl.BlockSpec((d, hh), lambda i: (0, 0)),
            pl.BlockSpec((1, hh), lambda i: (0, 0)),
        ],
        out_specs=[
            pl.BlockSpec((block, hh), lambda i: (i, 0)),
            pl.BlockSpec((block, hh), lambda i: (i, 0)),
        ],
        out_shape=[
            jax.ShapeDtypeStruct((n, hh), jnp.float32),
            jax.ShapeDtypeStruct((n, hh), jnp.float32),
        ],
    )(h, w1s, w1t, b1)


def _make_gather(n, e, hh):
    c_total = e // CHUNK
    base = c_total // NW
    extra = c_total % NW

    @functools.partial(
        pl.kernel,
        out_type=(
            jax.ShapeDtypeStruct((e, hh), jnp.float32),
            jax.ShapeDtypeStruct((e, hh), jnp.float32),
            jax.ShapeDtypeStruct((e,), jnp.float32),
        ),
        mesh=_sc_mesh(),
        compiler_params=pltpu.CompilerParams(needs_layout_passes=False),
        scratch_types=[
            pltpu.VMEM((n,), jnp.float32),
            pltpu.VMEM((n,), jnp.float32),
            pltpu.VMEM((n,), jnp.float32),
            pltpu.VMEM((CHUNK,), jnp.int32),
            pltpu.VMEM((CHUNK,), jnp.int32),
            pltpu.VMEM((CHUNK, hh), jnp.float32),
            pltpu.VMEM((CHUNK, hh), jnp.float32),
            pltpu.VMEM((CHUNK,), jnp.float32),
            pltpu.SemaphoreType.DMA,
            pltpu.SemaphoreType.DMA,
        ],
    )
    def gather_kernel(a_hbm, b_hbm, row_hbm, col_hbm, cx_hbm, cy_hbm, cz_hbm,
                      ar_hbm, bc_hbm, rad_hbm,
                      cx_v, cy_v, cz_v, idxr_v, idxc_v, a_v, b_v, rad_v,
                      sem_a, sem_b):
        cid = lax.axis_index("c")
        sid = lax.axis_index("s")
        wid = sid * NUM_CORES + cid
        pltpu.sync_copy(cx_hbm, cx_v)
        pltpu.sync_copy(cy_hbm, cy_v)
        pltpu.sync_copy(cz_hbm, cz_v)
        start = wid * base + jnp.minimum(wid, extra)
        count = base + (wid < extra).astype(jnp.int32)

        def body(i, carry):
            g = (start + i) * CHUNK
            pltpu.sync_copy(row_hbm.at[pl.ds(g, CHUNK)], idxr_v)
            pltpu.sync_copy(col_hbm.at[pl.ds(g, CHUNK)], idxc_v)
            cpa = pltpu.async_copy(a_hbm.at[idxr_v], a_v, sem_a)
            cpb = pltpu.async_copy(b_hbm.at[idxc_v], b_v, sem_b)
            for j in range(CHUNK // LANES):
                s = pl.ds(j * LANES, LANES)
                ir = idxr_v[s]
                ic = idxc_v[s]
                dx = plsc.load_gather(cx_v, [ir]) - plsc.load_gather(cx_v, [ic])
                dy = plsc.load_gather(cy_v, [ir]) - plsc.load_gather(cy_v, [ic])
                dz = plsc.load_gather(cz_v, [ir]) - plsc.load_gather(cz_v, [ic])
                rad_v[s] = dx * dx + dy * dy + dz * dz
            pltpu.sync_copy(rad_v, rad_hbm.at[pl.ds(g, CHUNK)])
            cpa.wait()
            cpb.wait()
            pltpu.sync_copy(a_v, ar_hbm.at[pl.ds(g, CHUNK), :])
            pltpu.sync_copy(b_v, bc_hbm.at[pl.ds(g, CHUNK), :])
            return carry

        lax.fori_loop(0, count, body, 0)

    return gather_kernel


def _edge_body(ar_ref, bc_ref, rad_ref, w1r_ref, w2_ref, b2_ref, out_ref):
    x = ar_ref[...] + bc_ref[...] + rad_ref[...] * w1r_ref[...]
    x = x * jax.nn.sigmoid(x)
    y = jnp.dot(x, w2_ref[...], preferred_element_type=jnp.float32) + b2_ref[...]
    out_ref[...] = y * jax.nn.sigmoid(y)


def _edge_mlp(ar, bc, rad, w1r, w2, b2, block=2560):
    e, hh = ar.shape
    grid = e // block
    return pl.pallas_call(
        _edge_body,
        grid=(grid,),
        in_specs=[
            pl.BlockSpec((block, hh), lambda i: (i, 0)),
            pl.BlockSpec((block, hh), lambda i: (i, 0)),
            pl.BlockSpec((block, 1), lambda i: (i, 0)),
            pl.BlockSpec((1, hh), lambda i: (0, 0)),
            pl.BlockSpec((hh, hh), lambda i: (0, 0)),
            pl.BlockSpec((1, hh), lambda i: (0, 0)),
        ],
        out_specs=pl.BlockSpec((block, hh), lambda i: (i, 0)),
        out_shape=jax.ShapeDtypeStruct((e, hh), jnp.float32),
    )(ar, bc, rad, w1r, w2, b2)


def _make_scatter(n, e, hh):
    c_total = e // CHUNK
    base = c_total // NW
    extra = c_total % NW
    n_pad = ((n + NUM_SUBCORES * CHUNK - 1) // (NUM_SUBCORES * CHUNK)) * NUM_SUBCORES * CHUNK
    rows_per_tile = n_pad // NUM_SUBCORES
    zchunk = CHUNK
    nz = rows_per_tile // zchunk

    @functools.partial(
        pl.kernel,
        out_type=jax.ShapeDtypeStruct((NUM_CORES, n_pad, hh), jnp.float32),
        mesh=_sc_mesh(),
        compiler_params=pltpu.CompilerParams(needs_layout_passes=False),
        scratch_types=[
            pltpu.VMEM_SHARED((n_pad, hh), jnp.float32),
            pltpu.VMEM((1, CHUNK), jnp.int32),
            pltpu.VMEM((CHUNK, hh), jnp.float32),
            pltpu.VMEM((zchunk, hh), jnp.float32),
        ],
    )
    def scatter_kernel(ef_hbm, row_hbm, zero_hbm, out_hbm,
                       agg_sh, idx_v, ef_v, buf_v):
        cid = lax.axis_index("c")
        sid = lax.axis_index("s")
        wid = sid * NUM_CORES + cid
        pltpu.sync_copy(zero_hbm, buf_v)
        for k in range(nz):
            pltpu.sync_copy(
                buf_v, agg_sh.at[pl.ds(sid * rows_per_tile + k * zchunk, zchunk), :])
        plsc.subcore_barrier()
        start = wid * base + jnp.minimum(wid, extra)
        count = base + (wid < extra).astype(jnp.int32)

        def body(i, carry):
            g = (start + i) * CHUNK
            pltpu.sync_copy(row_hbm.at[pl.ds(g, CHUNK)], idx_v.at[0])
            pltpu.sync_copy(ef_hbm.at[pl.ds(g, CHUNK), :], ef_v)
            pltpu.sync_copy(ef_v, agg_sh.at[idx_v.at[0]], add=True)
            return carry

        lax.fori_loop(0, count, body, 0)
        plsc.subcore_barrier()
        for k in range(nz):
            r = sid * rows_per_tile + k * zchunk
            pltpu.sync_copy(agg_sh.at[pl.ds(r, zchunk), :], buf_v)
            pltpu.sync_copy(buf_v, out_hbm.at[cid, pl.ds(r, zchunk), :])

    return scatter_kernel


def _node_body(h_ref, p0_ref, p1_ref, wa_ref, wb_ref, b1_ref, w2_ref, b2_ref,
               o_ref):
    h = h_ref[...]
    agg = p0_ref[...] + p1_ref[...]
    z = (jnp.dot(h, wa_ref[...], preferred_element_type=jnp.float32)
         + jnp.dot(agg, wb_ref[...], preferred_element_type=jnp.float32)
         + b1_ref[...])
    z = z * jax.nn.sigmoid(z)
    o_ref[...] = h + jnp.dot(z, w2_ref[...], preferred_element_type=jnp.float32) + b2_ref[...]


def _node_mlp(h, p0, p1, wa, wb, b1, w2, b2, block=2000):
    n, d = h.shape
    hh = wa.shape[1]
    grid = n // block
    return pl.pallas_call(
        _node_body,
        grid=(grid,),
        in_specs=[
            pl.BlockSpec((block, d), lambda i: (i, 0)),
            pl.BlockSpec((block, hh), lambda i: (i, 0)),
            pl.BlockSpec((block, hh), lambda i: (i, 0)),
            pl.BlockSpec((d, hh), lambda i: (0, 0)),
            pl.BlockSpec((hh, hh), lambda i: (0, 0)),
            pl.BlockSpec((1, hh), lambda i: (0, 0)),
            pl.BlockSpec((hh, d), lambda i: (0, 0)),
            pl.BlockSpec((1, d), lambda i: (0, 0)),
        ],
        out_specs=pl.BlockSpec((block, d), lambda i: (i, 0)),
        out_shape=jax.ShapeDtypeStruct((n, d), jnp.float32),
    )(h, p0, p1, wa, wb, b1, w2, b2)


def kernel(h, edge_index, coord, W_e1, b_e1, W_e2, b_e2, W_n1, b_n1, W_n2, b_n2):
    n, d = h.shape
    e = edge_index.shape[1]
    hh = W_e2.shape[0]

    row = edge_index[0]
    col = edge_index[1]
    w1s = W_e1[:d]
    w1t = W_e1[d:2 * d]
    w1r = W_e1[2 * d:]

    a, b = _precompute(h, w1s, w1t, b_e1.reshape(1, hh))

    cx = coord[:, 0]
    cy = coord[:, 1]
    cz = coord[:, 2]
    ar, bc, rad = _make_gather(n, e, hh)(a, b, row, col, cx, cy, cz)

    ef = _edge_mlp(ar, bc, rad.reshape(e, 1), w1r, W_e2, b_e2.reshape(1, hh))

    zeros = jnp.zeros((CHUNK, hh), jnp.float32)
    parts = _make_scatter(n, e, hh)(ef, row, zeros)

    return _node_mlp(h, parts[0, :n], parts[1, :n], W_n1[:d], W_n1[d:],
                     b_n1.reshape(1, hh), W_n2, b_n2.reshape(1, d))
